# baseline (speedup 1.0000x reference)
import jax
import jax.numpy as jnp
from jax.experimental import pallas as pl
from jax.experimental.pallas import tpu as pltpu

HIDDEN = 2048
R1P = 9
TM = 512


def _body(x_ref, w1_ref, b1_ref, w2_ref, b2_ref, o_ref):
    x = x_ref[...].astype(jnp.bfloat16)
    h = jnp.dot(x, w1_ref[...], preferred_element_type=jnp.float32)
    h = h + b1_ref[...]
    h = 0.5 * h * (1.0 + jax.lax.erf(h * 0.7071067811865476))
    o = jnp.dot(h.astype(jnp.bfloat16), w2_ref[...],
                preferred_element_type=jnp.float32)
    o_ref[...] = o + b2_ref[...]


def kernel(hidden_states, W1, b1, W2, b2):
    tokens = hidden_states.shape[0]
    grid = (tokens // TM,)
    w1b = W1.astype(jnp.bfloat16)
    w2b = W2.astype(jnp.bfloat16)
    b1r = b1.reshape(1, HIDDEN)
    b2r = b2.reshape(1, R1P)
    return pl.pallas_call(
        _body,
        grid=grid,
        in_specs=[
            pl.BlockSpec((TM, HIDDEN), lambda i: (i, 0)),
            pl.BlockSpec((HIDDEN, HIDDEN), lambda i: (0, 0)),
            pl.BlockSpec((1, HIDDEN), lambda i: (0, 0)),
            pl.BlockSpec((HIDDEN, R1P), lambda i: (0, 0)),
            pl.BlockSpec((1, R1P), lambda i: (0, 0)),
        ],
        out_specs=pl.BlockSpec((TM, R1P), lambda i: (i, 0)),
        out_shape=jax.ShapeDtypeStruct((tokens, R1P), jnp.float32),
        compiler_params=pltpu.CompilerParams(
            dimension_semantics=("parallel",),
        ),
    )(hidden_states, w1b, b1r, w2b, b2r)


# speedup vs baseline: 1.0830x; 1.0830x over previous
import jax
import jax.numpy as jnp
from jax.experimental import pallas as pl
from jax.experimental.pallas import tpu as pltpu

HIDDEN = 2048
R1P = 9
TM = 512


def _body(x_ref, w1_ref, b1_ref, w2_ref, b2_ref, o_ref):
    h = jnp.dot(x_ref[...], w1_ref[...], preferred_element_type=jnp.float32)
    h = h + b1_ref[...]
    h = 0.5 * h * (1.0 + jax.lax.erf(h * 0.7071067811865476))
    o = jnp.dot(h, w2_ref[...], preferred_element_type=jnp.float32)
    o_ref[...] = o + b2_ref[...]


def kernel(hidden_states, W1, b1, W2, b2):
    tokens = hidden_states.shape[0]
    grid = (tokens // TM,)
    b1r = b1.reshape(1, HIDDEN)
    b2r = b2.reshape(1, R1P)
    return pl.pallas_call(
        _body,
        grid=grid,
        in_specs=[
            pl.BlockSpec((TM, HIDDEN), lambda i: (i, 0)),
            pl.BlockSpec((HIDDEN, HIDDEN), lambda i: (0, 0)),
            pl.BlockSpec((1, HIDDEN), lambda i: (0, 0)),
            pl.BlockSpec((HIDDEN, R1P), lambda i: (0, 0)),
            pl.BlockSpec((1, R1P), lambda i: (0, 0)),
        ],
        out_specs=pl.BlockSpec((TM, R1P), lambda i: (i, 0)),
        out_shape=jax.ShapeDtypeStruct((tokens, R1P), jnp.float32),
        compiler_params=pltpu.CompilerParams(
            dimension_semantics=("parallel",),
        ),
    )(hidden_states, W1, b1r, W2, b2r)


# speedup vs baseline: 1.0997x; 1.0155x over previous
import jax
import jax.numpy as jnp
from jax.experimental import pallas as pl
from jax.experimental.pallas import tpu as pltpu

HIDDEN = 2048
R1P = 9
TM = 1024


def _body(x_ref, w1_ref, b1_ref, w2_ref, b2_ref, o_ref):
    h = jnp.dot(x_ref[...], w1_ref[...], preferred_element_type=jnp.float32)
    h = h + b1_ref[...]
    h = 0.5 * h * (1.0 + jax.lax.erf(h * 0.7071067811865476))
    o = jnp.dot(h, w2_ref[...], preferred_element_type=jnp.float32)
    o_ref[...] = o + b2_ref[...]


def kernel(hidden_states, W1, b1, W2, b2):
    tokens = hidden_states.shape[0]
    grid = (tokens // TM,)
    b1r = b1.reshape(1, HIDDEN)
    b2r = b2.reshape(1, R1P)
    return pl.pallas_call(
        _body,
        grid=grid,
        in_specs=[
            pl.BlockSpec((TM, HIDDEN), lambda i: (i, 0)),
            pl.BlockSpec((HIDDEN, HIDDEN), lambda i: (0, 0)),
            pl.BlockSpec((1, HIDDEN), lambda i: (0, 0)),
            pl.BlockSpec((HIDDEN, R1P), lambda i: (0, 0)),
            pl.BlockSpec((1, R1P), lambda i: (0, 0)),
        ],
        out_specs=pl.BlockSpec((TM, R1P), lambda i: (i, 0)),
        out_shape=jax.ShapeDtypeStruct((tokens, R1P), jnp.float32),
        compiler_params=pltpu.CompilerParams(
            dimension_semantics=("parallel",),
        ),
    )(hidden_states, W1, b1r, W2, b2r)
